# Initial kernel scaffold; baseline (speedup 1.0000x reference)
import jax
import jax.numpy as jnp
from jax.experimental import pallas as pl


def kernel(X, Q, ranks):
    raise NotImplementedError("write your pallas kernel here")


# speedup vs baseline: 6.1317x; 6.1317x over previous
import jax
import jax.numpy as jnp
from jax import lax
from jax.experimental import pallas as pl
from jax.experimental.pallas import tpu as pltpu

_M = 400
_K = 10
_BETA = 0.15
_NEG = -1e30


def _tree_sum(terms):
    t = list(terms) + [None] * (16 - len(terms))

    def add(a, b):
        if a is None:
            return b
        if b is None:
            return a
        return a + b

    s1 = [add(t[i], t[i + 8]) for i in range(8)]
    s2 = [add(s1[i], s1[i + 4]) for i in range(4)]
    s3 = [add(s2[i], s2[i + 2]) for i in range(2)]
    return add(s3[0], s3[1])


def _combine_body(ranks_s_ref, x_ref, xdba_ref, xb_ref, s_ref):
    def _gather(j, carry):
        idx = ranks_s_ref[0, 0, j]
        xb_ref[pl.ds(j, 1), :] = x_ref[pl.ds(idx, 1), :]
        return carry
    lax.fori_loop(0, _M, _gather, 0)

    xb = xb_ref[...]
    s_ref[...] = lax.dot_general(xb, xb, (((1,), (1,)), ((), ())),
                                 preferred_element_type=jnp.float32)

    col = lax.broadcasted_iota(jnp.int32, (_M, _M), 1)

    terms = []
    wterms = []
    for k in range(_K):
        s = s_ref[...]
        maxv = jnp.max(s, axis=1, keepdims=True)
        idxk = jnp.min(jnp.where(s == maxv, col, _M), axis=1, keepdims=True)
        onehot = col == idxk
        w = jnp.ones_like(maxv) if k == 0 else maxv * _BETA
        gk = lax.dot_general(jnp.where(onehot, 1.0, 0.0), xb,
                             (((1,), (0,)), ((), ())),
                             preferred_element_type=jnp.float32,
                             precision=lax.Precision.HIGHEST)
        terms.append(w * gk)
        wterms.append(w)
        s_ref[...] = jnp.where(onehot, _NEG, s)

    xdba_ref[0] = _tree_sum(terms) / _tree_sum(wterms)


def _rank_body(xd_ref, q_ref, ranks_c_ref, final_ref, res_ref, pre_ref):
    xd = xd_ref[0]
    q = q_ref[0]
    resr = lax.dot_general(q[:, 0:128], xd[:, 0:128],
                           (((1,), (1,)), ((), ())),
                           preferred_element_type=jnp.float32)
    for c in range(1, 4):
        resr = resr + lax.dot_general(q[:, c * 128:(c + 1) * 128],
                                      xd[:, c * 128:(c + 1) * 128],
                                      (((1,), (1,)), ((), ())),
                                      preferred_element_type=jnp.float32)
    resc = jnp.transpose(resr)
    res_ref[0, 0, :] = resr[0, :]

    col = lax.broadcasted_iota(jnp.int32, (_M, _M), 1)
    row = lax.broadcasted_iota(jnp.int32, (_M, _M), 0)
    cmp = (resr > resc) | ((resr == resc) & (col < row))
    rank = jnp.sum(cmp.astype(jnp.int32), axis=1, keepdims=True)
    eq = rank == col
    pre = jnp.sum(jnp.where(eq, row, 0), axis=0)
    final = jnp.sum(jnp.where(eq, ranks_c_ref[0], 0), axis=0)
    pre_ref[0, 0, :] = pre
    final_ref[0, 0, :] = final


def kernel(X, Q, ranks):
    nq = ranks.shape[1]
    d = X.shape[1]
    ranks_t = jnp.transpose(ranks[:_M, :]).reshape(nq, 1, _M)
    ranks_c = ranks_t.reshape(nq, _M, 1)
    q3 = Q.reshape(nq, 1, d)

    xdba = pl.pallas_call(
        _combine_body,
        grid=(nq,),
        in_specs=[
            pl.BlockSpec((1, 1, _M), lambda i: (i, 0, 0),
                         memory_space=pltpu.SMEM),
            pl.BlockSpec(X.shape, lambda i: (0, 0)),
        ],
        out_specs=pl.BlockSpec((1, _M, d), lambda i: (i, 0, 0)),
        out_shape=jax.ShapeDtypeStruct((nq, _M, d), jnp.float32),
        scratch_shapes=[pltpu.VMEM((_M, d), jnp.float32),
                        pltpu.VMEM((_M, _M), jnp.float32)],
    )(ranks_t, X)

    final3, res3, pre3 = pl.pallas_call(
        _rank_body,
        grid=(nq,),
        in_specs=[
            pl.BlockSpec((1, _M, d), lambda i: (i, 0, 0)),
            pl.BlockSpec((1, 1, d), lambda i: (i, 0, 0)),
            pl.BlockSpec((1, _M, 1), lambda i: (i, 0, 0)),
        ],
        out_specs=[
            pl.BlockSpec((1, 1, _M), lambda i: (i, 0, 0)),
            pl.BlockSpec((1, 1, _M), lambda i: (i, 0, 0)),
            pl.BlockSpec((1, 1, _M), lambda i: (i, 0, 0)),
        ],
        out_shape=(
            jax.ShapeDtypeStruct((nq, 1, _M), jnp.int32),
            jax.ShapeDtypeStruct((nq, 1, _M), jnp.float32),
            jax.ShapeDtypeStruct((nq, 1, _M), jnp.int32),
        ),
    )(xdba, q3, ranks_c)

    return (final3.reshape(nq, _M), res3.reshape(nq, _M),
            pre3.reshape(nq, _M), xdba)


# speedup vs baseline: 9.6119x; 1.5676x over previous
import jax
import jax.numpy as jnp
from jax import lax
from jax.experimental import pallas as pl
from jax.experimental.pallas import tpu as pltpu

_M = 400
_K = 10
_BETA = 0.15
_NEG = -1e30


def _tree_sum(terms):
    t = list(terms) + [None] * (16 - len(terms))

    def add(a, b):
        if a is None:
            return b
        if b is None:
            return a
        return a + b

    s1 = [add(t[i], t[i + 8]) for i in range(8)]
    s2 = [add(s1[i], s1[i + 4]) for i in range(4)]
    s3 = [add(s2[i], s2[i + 2]) for i in range(2)]
    return add(s3[0], s3[1])


def _combine_body(ranks_s_ref, x_ref, xdba_ref, xb_ref, s_ref):
    def _gather(j, carry):
        idx = ranks_s_ref[0, 0, j]
        xb_ref[pl.ds(j, 1), :] = x_ref[pl.ds(idx, 1), :]
        return carry
    lax.fori_loop(0, _M, _gather, 0)

    xb = xb_ref[...]
    s_ref[...] = lax.dot_general(xb, xb, (((1,), (1,)), ((), ())),
                                 preferred_element_type=jnp.float32)

    col = lax.broadcasted_iota(jnp.int32, (_M, _M), 1)

    hi = xb.astype(jnp.bfloat16).astype(jnp.float32)
    r1 = xb - hi
    mid = r1.astype(jnp.bfloat16).astype(jnp.float32)
    lo = r1 - mid

    terms = []
    wterms = []
    for k in range(_K):
        s = s_ref[...]
        maxv = jnp.max(s, axis=1, keepdims=True)
        idxk = jnp.min(jnp.where(s == maxv, col, _M), axis=1, keepdims=True)
        onehot = col == idxk
        w = jnp.ones_like(maxv) if k == 0 else maxv * _BETA
        oh = jnp.where(onehot, 1.0, 0.0)
        gk = (lax.dot_general(oh, hi, (((1,), (0,)), ((), ())),
                              preferred_element_type=jnp.float32)
              + lax.dot_general(oh, mid, (((1,), (0,)), ((), ())),
                                preferred_element_type=jnp.float32))
        gk = gk + lax.dot_general(oh, lo, (((1,), (0,)), ((), ())),
                                  preferred_element_type=jnp.float32)
        terms.append(w * gk)
        wterms.append(w)
        s_ref[...] = jnp.where(onehot, _NEG, s)

    xdba_ref[0] = _tree_sum(terms) / _tree_sum(wterms)


def _rank_body(xd_ref, q_ref, ranks_c_ref, final_ref, res_ref, pre_ref):
    xd = xd_ref[0]
    q = q_ref[0]
    resr = lax.dot_general(q[:, 0:128], xd[:, 0:128],
                           (((1,), (1,)), ((), ())),
                           preferred_element_type=jnp.float32)
    for c in range(1, 4):
        resr = resr + lax.dot_general(q[:, c * 128:(c + 1) * 128],
                                      xd[:, c * 128:(c + 1) * 128],
                                      (((1,), (1,)), ((), ())),
                                      preferred_element_type=jnp.float32)
    resc = jnp.transpose(resr)
    res_ref[0, 0, :] = resr[0, :]

    col = lax.broadcasted_iota(jnp.int32, (_M, _M), 1)
    row = lax.broadcasted_iota(jnp.int32, (_M, _M), 0)
    cmp = (resr > resc) | ((resr == resc) & (col < row))
    rank = jnp.sum(cmp.astype(jnp.int32), axis=1, keepdims=True)
    eq = rank == col
    pre = jnp.sum(jnp.where(eq, row, 0), axis=0)
    final = jnp.sum(jnp.where(eq, ranks_c_ref[0], 0), axis=0)
    pre_ref[0, 0, :] = pre
    final_ref[0, 0, :] = final


def kernel(X, Q, ranks):
    nq = ranks.shape[1]
    d = X.shape[1]
    ranks_t = jnp.transpose(ranks[:_M, :]).reshape(nq, 1, _M)
    ranks_c = ranks_t.reshape(nq, _M, 1)
    q3 = Q.reshape(nq, 1, d)

    xdba = pl.pallas_call(
        _combine_body,
        grid=(nq,),
        in_specs=[
            pl.BlockSpec((1, 1, _M), lambda i: (i, 0, 0),
                         memory_space=pltpu.SMEM),
            pl.BlockSpec(X.shape, lambda i: (0, 0)),
        ],
        out_specs=pl.BlockSpec((1, _M, d), lambda i: (i, 0, 0)),
        out_shape=jax.ShapeDtypeStruct((nq, _M, d), jnp.float32),
        scratch_shapes=[pltpu.VMEM((_M, d), jnp.float32),
                        pltpu.VMEM((_M, _M), jnp.float32)],
    )(ranks_t, X)

    final3, res3, pre3 = pl.pallas_call(
        _rank_body,
        grid=(nq,),
        in_specs=[
            pl.BlockSpec((1, _M, d), lambda i: (i, 0, 0)),
            pl.BlockSpec((1, 1, d), lambda i: (i, 0, 0)),
            pl.BlockSpec((1, _M, 1), lambda i: (i, 0, 0)),
        ],
        out_specs=[
            pl.BlockSpec((1, 1, _M), lambda i: (i, 0, 0)),
            pl.BlockSpec((1, 1, _M), lambda i: (i, 0, 0)),
            pl.BlockSpec((1, 1, _M), lambda i: (i, 0, 0)),
        ],
        out_shape=(
            jax.ShapeDtypeStruct((nq, 1, _M), jnp.int32),
            jax.ShapeDtypeStruct((nq, 1, _M), jnp.float32),
            jax.ShapeDtypeStruct((nq, 1, _M), jnp.int32),
        ),
    )(xdba, q3, ranks_c)

    return (final3.reshape(nq, _M), res3.reshape(nq, _M),
            pre3.reshape(nq, _M), xdba)
